# baseline (speedup 1.0000x reference)
import functools

import jax
import jax.numpy as jnp
from jax import lax
from jax.experimental import pallas as pl
from jax.experimental.pallas import tpu as pltpu
from jax.experimental.pallas import tpu_sc as plsc

N = 50000
NP = 51200
E = 1600000
EP = 1605632
NCH = EP // 128
G = 512
GP = 1024
D = 60
DP = 64
HC = 32
SK = 4
NB = 3
LA = 1
NCP = NCH // 2
EPS = 1e-5

_mesh = plsc.VectorSubcoreMesh(core_axis_name="c", subcore_axis_name="s")
_sc_params = pltpu.CompilerParams(needs_layout_passes=False,
                                  use_tc_tiling_on_sc=False)


def _deg_body(dst2d, batch2d, znp, deg_out, cnt_out, didx, bidx, deg_acc,
              cnt_acc):
    cid = lax.axis_index("c")
    sid = lax.axis_index("s")
    wid = cid * 16 + sid
    ones16 = jnp.ones((16,), jnp.float32)
    pltpu.sync_copy(znp.at[pl.ds(0, NP)], deg_acc)

    cpt = NCH // 32

    def step(s, carry):
        base = wid * cpt + s * 8
        pltpu.sync_copy(dst2d.at[pl.ds(base, 8)], didx)
        for j in range(8):
            for k in range(8):
                vec = didx[j, pl.ds(k * 16, 16)]
                plsc.addupdate_scatter(deg_acc, [vec], ones16)
        return carry

    lax.fori_loop(0, cpt // 8, step, 0)
    pltpu.sync_copy(deg_acc, deg_out.at[wid])

    @pl.when(cid == 0)
    def _():
        pltpu.sync_copy(znp.at[pl.ds(0, GP)], cnt_acc)

        def cstep(c, carry):
            pltpu.sync_copy(batch2d.at[pl.ds(sid * 25 + c, 1)], bidx)
            for k in range(8):
                vec = bidx[0, pl.ds(k * 16, 16)]
                plsc.addupdate_scatter(cnt_acc, [vec], ones16)
            return carry

        lax.fori_loop(0, 25, cstep, 0)
        pltpu.sync_copy(cnt_acc, cnt_out.at[sid])


_sck_deg = pl.kernel(
    _deg_body,
    out_type=(jax.ShapeDtypeStruct((32, NP), jnp.float32),
              jax.ShapeDtypeStruct((16, GP), jnp.float32)),
    mesh=_mesh,
    compiler_params=_sc_params,
    scratch_types=[
        pltpu.VMEM((8, 128), jnp.int32),
        pltpu.VMEM((1, 128), jnp.int32),
        pltpu.VMEM((NP,), jnp.float32),
        pltpu.VMEM((GP,), jnp.float32),
    ],
)


def _agg_loop(table, sidx_src, dst2d, rows, sidx, didx, gsem, ssem, acc,
              cpt, base_of):
    nsteps = cpt // SK

    def drain(s, b, guarded):
        w = lambda: pltpu.make_async_copy(
            rows.at[b], acc.at[didx.at[0, 0]], ssem.at[b]).wait()
        if guarded:
            pl.when(s > 0)(w)
        else:
            w()

    def fire_gather(j, b):
        pltpu.async_copy(table.at[sidx.at[j]], rows.at[b], gsem.at[b])

    def step(s, carry):
        p = s % 2
        base = base_of(s)
        pltpu.sync_copy(sidx_src(base), sidx)
        pltpu.sync_copy(dst2d.at[pl.ds(base, SK)], didx.at[p])
        for j in range(LA + 1):
            drain(s, j % NB, True)
            fire_gather(j, j % NB)
        for j in range(SK):
            b = j % NB
            nj = j + LA + 1
            if nj < SK:
                b1 = nj % NB
                drain(s, b1, nj < NB)
                fire_gather(nj, b1)
            pltpu.make_async_copy(
                table.at[sidx.at[j]], rows.at[b], gsem.at[b]).wait()
            pltpu.async_copy(rows.at[b], acc.at[didx.at[p, j]], ssem.at[b],
                             add=True)
        return carry

    lax.fori_loop(0, nsteps, step, 0)
    for b in range(NB):
        pltpu.make_async_copy(rows.at[b], acc.at[didx.at[0, 0]],
                              ssem.at[b]).wait()


def _agg16_body(table, zt, src2d, dst2d, out, sidx, didx, rows, acc, gsem,
                ssem):
    cid = lax.axis_index("c")
    sid = lax.axis_index("s")
    wid = cid * 16 + sid
    stripe = NP // 16

    @pl.when(cid == 0)
    def _():
        pltpu.sync_copy(table.at[pl.ds(sid * stripe, stripe)],
                        acc.at[pl.ds(sid * stripe, stripe)])

    @pl.when(cid == 1)
    def _():
        pltpu.sync_copy(zt.at[pl.ds(sid * stripe, stripe)],
                        acc.at[pl.ds(sid * stripe, stripe)])

    plsc.subcore_barrier()
    cpt = NCP // 32
    _agg_loop(table, lambda base: src2d.at[pl.ds(base, SK)], dst2d, rows,
              sidx, didx, gsem, ssem, acc, cpt,
              lambda s: wid * cpt + s * SK)
    plsc.subcore_barrier()
    pltpu.sync_copy(acc.at[pl.ds(sid * stripe, stripe)],
                    out.at[pl.ds(cid * NP + sid * stripe, stripe)])


_sck_agg16 = pl.kernel(
    _agg16_body,
    out_type=jax.ShapeDtypeStruct((2 * NP, 16), jnp.float32),
    mesh=_mesh,
    compiler_params=_sc_params,
    scratch_types=[
        pltpu.VMEM((SK, 256), jnp.int32),
        pltpu.VMEM((2, SK, 256), jnp.int32),
        pltpu.VMEM((NB, 256, 16), jnp.float32),
        pltpu.VMEM_SHARED((NP, 16), jnp.float32),
        pltpu.SemaphoreType.DMA((NB,)),
        pltpu.SemaphoreType.DMA((NB,)),
    ],
)


def _agg32_body(table, srcp, dst2d, out, sidx, didx, rows, acc, gsem, ssem):
    cid = lax.axis_index("c")
    sid = lax.axis_index("s")
    stripe = NP // 16
    pltpu.sync_copy(table.at[pl.ds(cid * NP + sid * stripe, stripe)],
                    acc.at[pl.ds(sid * stripe, stripe)])
    plsc.subcore_barrier()
    cpt = NCP // 16
    _agg_loop(table, lambda base: srcp.at[cid, pl.ds(base, SK)], dst2d, rows,
              sidx, didx, gsem, ssem, acc, cpt,
              lambda s: sid * cpt + s * SK)
    plsc.subcore_barrier()
    pltpu.sync_copy(acc.at[pl.ds(sid * stripe, stripe)],
                    out.at[pl.ds(cid * NP + sid * stripe, stripe)])


_sck_agg32 = pl.kernel(
    _agg32_body,
    out_type=jax.ShapeDtypeStruct((2 * NP, HC), jnp.float32),
    mesh=_mesh,
    compiler_params=_sc_params,
    scratch_types=[
        pltpu.VMEM((SK, 256), jnp.int32),
        pltpu.VMEM((2, SK, 256), jnp.int32),
        pltpu.VMEM((NB, 256, HC), jnp.float32),
        pltpu.VMEM_SHARED((NP, HC), jnp.float32),
        pltpu.SemaphoreType.DMA((NB,)),
        pltpu.SemaphoreType.DMA((NB,)),
    ],
)


def _pool_body(h3tbl, batch2d, zg, out, bidx, rows, acc):
    cid = lax.axis_index("c")
    sid = lax.axis_index("s")
    gstripe = GP // 16
    pltpu.sync_copy(zg.at[pl.ds(sid * gstripe, gstripe)],
                    acc.at[pl.ds(sid * gstripe, gstripe)])
    plsc.subcore_barrier()

    def step(c, carry):
        ch = sid * 25 + c
        pltpu.sync_copy(batch2d.at[pl.ds(ch, 1)], bidx)
        pltpu.sync_copy(h3tbl.at[pl.ds(cid * NP + ch * 128, 128)], rows)
        pltpu.sync_copy(rows, acc.at[bidx.at[0]], add=True)
        return carry

    lax.fori_loop(0, 25, step, 0)
    plsc.subcore_barrier()
    pltpu.sync_copy(acc.at[pl.ds(sid * gstripe, gstripe)],
                    out.at[pl.ds(cid * GP + sid * gstripe, gstripe)])


_sck_pool = pl.kernel(
    _pool_body,
    out_type=jax.ShapeDtypeStruct((2 * GP, HC), jnp.float32),
    mesh=_mesh,
    compiler_params=_sc_params,
    scratch_types=[
        pltpu.VMEM((1, 128), jnp.int32),
        pltpu.VMEM((128, HC), jnp.float32),
        pltpu.VMEM_SHARED((GP, HC), jnp.float32),
    ],
)


R = 512


def _tck1_body(dp_ref, xp_ref, dinv_ref, g1_ref):
    deg = jnp.sum(dp_ref[...], axis=0) + 1.0
    dv = lax.rsqrt(deg)
    dinv_ref[...] = dv[:, None]
    g1_ref[...] = xp_ref[...] * dv[:, None]


def _tck1(deg_parts, xp):
    return pl.pallas_call(
        _tck1_body,
        grid=(NP // R,),
        in_specs=[
            pl.BlockSpec((32, R), lambda i: (0, i)),
            pl.BlockSpec((R, 16), lambda i: (i, 0)),
        ],
        out_specs=[
            pl.BlockSpec((R, 1), lambda i: (i, 0)),
            pl.BlockSpec((R, 16), lambda i: (i, 0)),
        ],
        out_shape=[
            jax.ShapeDtypeStruct((NP, 1), jnp.float32),
            jax.ShapeDtypeStruct((NP, 16), jnp.float32),
        ],
    )(deg_parts, xp)


def _tck2_body(ag_ref, dv_ref, w_ref, b_ref, o_ref):
    a = ag_ref[0] + ag_ref[1]
    dv = dv_ref[...]
    h = jnp.maximum(
        jnp.dot(a * dv, w_ref[...], preferred_element_type=jnp.float32)
        + b_ref[...], 0.0)
    g = h * dv
    o_ref[0] = g[:, :HC]
    o_ref[1] = g[:, HC:]


def _tck2(agg1, dinv, w1p, b1p):
    return pl.pallas_call(
        _tck2_body,
        grid=(NP // R,),
        in_specs=[
            pl.BlockSpec((2, R, 16), lambda i: (0, i, 0)),
            pl.BlockSpec((R, 1), lambda i: (i, 0)),
            pl.BlockSpec((16, DP), lambda i: (0, 0)),
            pl.BlockSpec((1, DP), lambda i: (0, 0)),
        ],
        out_specs=pl.BlockSpec((2, R, HC), lambda i: (0, i, 0)),
        out_shape=jax.ShapeDtypeStruct((2, NP, HC), jnp.float32),
    )(agg1, dinv, w1p, b1p)


def _tck3_body(final, ag_ref, dv_ref, w_ref, b_ref, o_ref):
    hin = jnp.concatenate([ag_ref[0], ag_ref[1]], axis=1)
    dv = dv_ref[...]
    h = jnp.maximum(
        jnp.dot(hin * dv, w_ref[...], preferred_element_type=jnp.float32)
        + b_ref[...], 0.0)
    g = h if final else h * dv
    o_ref[0] = g[:, :HC]
    o_ref[1] = g[:, HC:]


def _tck3(agg, dinv, wp, bp, final):
    return pl.pallas_call(
        functools.partial(_tck3_body, final),
        grid=(NP // R,),
        in_specs=[
            pl.BlockSpec((2, R, HC), lambda i: (0, i, 0)),
            pl.BlockSpec((R, 1), lambda i: (i, 0)),
            pl.BlockSpec((DP, DP), lambda i: (0, 0)),
            pl.BlockSpec((1, DP), lambda i: (0, 0)),
        ],
        out_specs=pl.BlockSpec((2, R, HC), lambda i: (0, i, 0)),
        out_shape=jax.ShapeDtypeStruct((2, NP, HC), jnp.float32),
    )(agg, dinv, wp, bp)


def _head_body(pool_ref, cp_ref, bn2_ref, linw_ref, linb_ref, bn3_ref,
               lin2w_ref, lin2b_ref, lin3w_ref, lin3b_ref, o_ref):
    counts = jnp.sum(cp_ref[...], axis=0)[:, None]
    invc = 1.0 / jnp.maximum(counts, 1.0)
    hg = jnp.concatenate([pool_ref[0], pool_ref[1]], axis=1) * invc
    bn2 = bn2_ref[...]
    s2 = bn2[0] * lax.rsqrt(bn2[3] + EPS)
    hg = (hg - bn2[2]) * s2 + bn2[1]
    h = jnp.maximum(
        jnp.dot(hg, linw_ref[...], preferred_element_type=jnp.float32)
        + linb_ref[...], 0.0)
    bn3 = bn3_ref[...]
    s3 = bn3[0] * lax.rsqrt(bn3[3] + EPS)
    h = (h - bn3[2]) * s3 + bn3[1]
    h = jnp.maximum(
        jnp.dot(h, lin2w_ref[...], preferred_element_type=jnp.float32)
        + lin2b_ref[...], 0.0)
    o = (jnp.dot(h, lin3w_ref[...], preferred_element_type=jnp.float32)
         + lin3b_ref[...])
    m = jnp.max(o, axis=1, keepdims=True)
    z = o - m
    lse = jnp.log(jnp.sum(jnp.exp(z), axis=1, keepdims=True))
    o_ref[...] = z - lse


def _tck_head(pool, cparts, bn2, linw, linb, bn3, lin2w, lin2b, lin3w, lin3b):
    full = lambda s: pl.BlockSpec(s, lambda: tuple(0 for _ in s))
    return pl.pallas_call(
        _head_body,
        in_specs=[
            full((2, G, HC)), full((16, G)), full((4, DP)), full((DP, DP)),
            full((1, DP)), full((4, DP)), full((DP, DP)), full((1, DP)),
            full((DP, DP)), full((1, DP)),
        ],
        out_specs=full((G, DP)),
        out_shape=jax.ShapeDtypeStruct((G, DP), jnp.float32),
    )(pool[:, :G, :], cparts[:, :G], bn2, linw, linb, bn3, lin2w, lin2b,
      lin3w, lin3b)


def _pad_vec(v, fill=0.0):
    return jnp.pad(v, (0, DP - v.shape[0]), constant_values=fill)[None, :]


def kernel(x, edge_index, batch, W1, b1, W2, b2, W3, b3, bn2_g, bn2_b, bn2_m,
           bn2_v, bn3_g, bn3_b, bn3_m, bn3_v, lin_W, lin_b, lin2_W, lin2_b,
           lin3_W, lin3_b):
    src = jnp.concatenate(
        [edge_index[0], jnp.full((EP - E,), N, jnp.int32)]).reshape(NCH, 128)
    dst = jnp.concatenate(
        [edge_index[1], jnp.full((EP - E,), N, jnp.int32)]).reshape(NCH, 128)
    srcq = src.reshape(NCP, 256)
    dstq = dst.reshape(NCP, 256)
    srcp = jnp.stack([srcq, srcq + NP])
    batch2d = jnp.concatenate(
        [batch, jnp.full((NP - N,), G, jnp.int32)]).reshape(NP // 128, 128)
    xp = jnp.pad(x, ((0, NP - N), (0, 0)))
    znp = jnp.zeros((NP,), jnp.float32)
    zt16 = jnp.zeros((NP, 16), jnp.float32)
    zg = jnp.zeros((GP, HC), jnp.float32)
    w1p = jnp.pad(W1, ((0, 0), (0, DP - D)))
    w2p = jnp.pad(W2, ((0, DP - D), (0, DP - D)))
    w3p = jnp.pad(W3, ((0, DP - D), (0, DP - D)))
    linwp = jnp.pad(lin_W, ((0, DP - D), (0, DP - D)))
    lin2wp = jnp.pad(lin2_W, ((0, DP - D), (0, DP - D)))
    lin3wp = jnp.pad(lin3_W, ((0, DP - D), (0, DP - 4)))
    b1p, b2p, b3p = _pad_vec(b1), _pad_vec(b2), _pad_vec(b3)
    linbp, lin2bp = _pad_vec(lin_b), _pad_vec(lin2_b)
    lin3bp = _pad_vec(lin3_b, fill=-1e30)
    bn2 = jnp.stack([
        _pad_vec(bn2_g)[0], _pad_vec(bn2_b)[0], _pad_vec(bn2_m)[0],
        _pad_vec(bn2_v, fill=1.0)[0]])
    bn3 = jnp.stack([
        _pad_vec(bn3_g)[0], _pad_vec(bn3_b)[0], _pad_vec(bn3_m)[0],
        _pad_vec(bn3_v, fill=1.0)[0]])

    deg_parts, cnt_parts = _sck_deg(dst, batch2d, znp)
    dinv, g1 = _tck1(deg_parts, xp)

    agg1 = _sck_agg16(g1, zt16, srcq, dstq).reshape(2, NP, 16)
    g2 = _tck2(agg1, dinv, w1p, b1p)
    agg2 = _sck_agg32(g2.reshape(2 * NP, HC), srcp, dstq).reshape(2, NP, HC)
    g3 = _tck3(agg2, dinv, w2p, b2p, final=False)
    agg3 = _sck_agg32(g3.reshape(2 * NP, HC), srcp, dstq).reshape(2, NP, HC)
    h3 = _tck3(agg3, dinv, w3p, b3p, final=True)

    pool = _sck_pool(h3.reshape(2 * NP, HC), batch2d, zg).reshape(2, GP, HC)
    out = _tck_head(pool, cnt_parts, bn2, linwp, linbp, bn3, lin2wp, lin2bp,
                    lin3wp, lin3bp)
    return out[:, :4]


# speedup vs baseline: 1.2251x; 1.2251x over previous
import functools

import jax
import jax.numpy as jnp
from jax import lax
from jax.experimental import pallas as pl
from jax.experimental.pallas import tpu as pltpu
from jax.experimental.pallas import tpu_sc as plsc

N = 50000
NP = 51200
E = 1600000
EP = 1605632
NCH = EP // 128
G = 512
GP = 1024
D = 60
DP = 64
HC = 32
SK = 8
NB = 5
LA = 2
EPS = 1e-5

_mesh = plsc.VectorSubcoreMesh(core_axis_name="c", subcore_axis_name="s")
_sc_params = pltpu.CompilerParams(needs_layout_passes=False,
                                  use_tc_tiling_on_sc=False)


def _deg_body(dst2d, batch2d, znp, deg_out, cnt_out, didx, bidx, deg_acc,
              cnt_acc):
    cid = lax.axis_index("c")
    sid = lax.axis_index("s")
    wid = cid * 16 + sid
    ones16 = jnp.ones((16,), jnp.float32)
    pltpu.sync_copy(znp.at[pl.ds(0, NP)], deg_acc)

    cpt = NCH // 32

    def step(s, carry):
        base = wid * cpt + s * SK
        pltpu.sync_copy(dst2d.at[pl.ds(base, SK)], didx)
        for j in range(SK):
            for k in range(8):
                vec = didx[j, pl.ds(k * 16, 16)]
                plsc.addupdate_scatter(deg_acc, [vec], ones16)
        return carry

    lax.fori_loop(0, cpt // SK, step, 0)
    pltpu.sync_copy(deg_acc, deg_out.at[wid])

    @pl.when(cid == 0)
    def _():
        pltpu.sync_copy(znp.at[pl.ds(0, GP)], cnt_acc)

        def cstep(c, carry):
            pltpu.sync_copy(batch2d.at[pl.ds(sid * 25 + c, 1)], bidx)
            for k in range(8):
                vec = bidx[0, pl.ds(k * 16, 16)]
                plsc.addupdate_scatter(cnt_acc, [vec], ones16)
            return carry

        lax.fori_loop(0, 25, cstep, 0)
        pltpu.sync_copy(cnt_acc, cnt_out.at[sid])


_sck_deg = pl.kernel(
    _deg_body,
    out_type=(jax.ShapeDtypeStruct((32, NP), jnp.float32),
              jax.ShapeDtypeStruct((16, GP), jnp.float32)),
    mesh=_mesh,
    compiler_params=_sc_params,
    scratch_types=[
        pltpu.VMEM((SK, 128), jnp.int32),
        pltpu.VMEM((1, 128), jnp.int32),
        pltpu.VMEM((NP,), jnp.float32),
        pltpu.VMEM((GP,), jnp.float32),
    ],
)


def _agg_loop(table, sidx_src, dst2d, rows, sidx, didx, gsem, ssem, isem,
              acc, cpt, base_of):
    nsteps = cpt // SK

    def drain(s, b, guarded):
        w = lambda: pltpu.make_async_copy(
            rows.at[b], acc.at[didx.at[0, 0]], ssem.at[b]).wait()
        if guarded:
            pl.when(s > 0)(w)
        else:
            w()

    def fire_idx(s):
        q = s % 2
        pltpu.async_copy(sidx_src(base_of(s)), sidx.at[q], isem.at[q])
        pltpu.async_copy(dst2d.at[pl.ds(base_of(s), SK)], didx.at[s % 3],
                         isem.at[q])

    def fire_gather(p2, j, b):
        pltpu.async_copy(table.at[sidx.at[p2, j]], rows.at[b], gsem.at[b])

    def step(s, carry):
        p2 = s % 2
        p3 = s % 3

        @pl.when(s + 1 < nsteps)
        def _():
            fire_idx(s + 1)

        pltpu.make_async_copy(sidx_src(base_of(s)), sidx.at[p2],
                              isem.at[p2]).wait()
        pltpu.make_async_copy(dst2d.at[pl.ds(base_of(s), SK)], didx.at[p3],
                              isem.at[p2]).wait()
        for j in range(LA + 1):
            drain(s, j % NB, True)
            fire_gather(p2, j, j % NB)
        for j in range(SK):
            b = j % NB
            nj = j + LA + 1
            if nj < SK:
                b1 = nj % NB
                drain(s, b1, nj < NB)
                fire_gather(p2, nj, b1)
            pltpu.make_async_copy(
                table.at[sidx.at[p2, j]], rows.at[b], gsem.at[b]).wait()
            pltpu.async_copy(rows.at[b], acc.at[didx.at[p3, j]], ssem.at[b],
                             add=True)
        return carry

    fire_idx(0)
    lax.fori_loop(0, nsteps, step, 0)
    for b in range(NB):
        pltpu.make_async_copy(rows.at[b], acc.at[didx.at[0, 0]],
                              ssem.at[b]).wait()


def _agg16_body(table, zt, src2d, dst2d, out, sidx, didx, rows, acc, gsem,
                ssem, isem):
    cid = lax.axis_index("c")
    sid = lax.axis_index("s")
    wid = cid * 16 + sid
    stripe = NP // 16

    @pl.when(cid == 0)
    def _():
        pltpu.sync_copy(table.at[pl.ds(sid * stripe, stripe)],
                        acc.at[pl.ds(sid * stripe, stripe)])

    @pl.when(cid == 1)
    def _():
        pltpu.sync_copy(zt.at[pl.ds(sid * stripe, stripe)],
                        acc.at[pl.ds(sid * stripe, stripe)])

    plsc.subcore_barrier()
    cpt = NCH // 32
    _agg_loop(table, lambda base: src2d.at[pl.ds(base, SK)], dst2d, rows,
              sidx, didx, gsem, ssem, isem, acc, cpt,
              lambda s: wid * cpt + s * SK)
    plsc.subcore_barrier()
    pltpu.sync_copy(acc.at[pl.ds(sid * stripe, stripe)],
                    out.at[pl.ds(cid * NP + sid * stripe, stripe)])


_sck_agg16 = pl.kernel(
    _agg16_body,
    out_type=jax.ShapeDtypeStruct((2 * NP, 16), jnp.float32),
    mesh=_mesh,
    compiler_params=_sc_params,
    scratch_types=[
        pltpu.VMEM((2, SK, 128), jnp.int32),
        pltpu.VMEM((3, SK, 128), jnp.int32),
        pltpu.VMEM((NB, 128, 16), jnp.float32),
        pltpu.VMEM_SHARED((NP, 16), jnp.float32),
        pltpu.SemaphoreType.DMA((NB,)),
        pltpu.SemaphoreType.DMA((NB,)),
        pltpu.SemaphoreType.DMA((2,)),
    ],
)


def _agg32_body(table, srcp, dst2d, out, sidx, didx, rows, acc, gsem, ssem,
                isem):
    cid = lax.axis_index("c")
    sid = lax.axis_index("s")
    stripe = NP // 16
    pltpu.sync_copy(table.at[pl.ds(cid * NP + sid * stripe, stripe)],
                    acc.at[pl.ds(sid * stripe, stripe)])
    plsc.subcore_barrier()
    cpt = NCH // 16
    _agg_loop(table, lambda base: srcp.at[cid, pl.ds(base, SK)], dst2d, rows,
              sidx, didx, gsem, ssem, isem, acc, cpt,
              lambda s: sid * cpt + s * SK)
    plsc.subcore_barrier()
    pltpu.sync_copy(acc.at[pl.ds(sid * stripe, stripe)],
                    out.at[pl.ds(cid * NP + sid * stripe, stripe)])


_sck_agg32 = pl.kernel(
    _agg32_body,
    out_type=jax.ShapeDtypeStruct((2 * NP, HC), jnp.float32),
    mesh=_mesh,
    compiler_params=_sc_params,
    scratch_types=[
        pltpu.VMEM((2, SK, 128), jnp.int32),
        pltpu.VMEM((3, SK, 128), jnp.int32),
        pltpu.VMEM((NB, 128, HC), jnp.float32),
        pltpu.VMEM_SHARED((NP, HC), jnp.float32),
        pltpu.SemaphoreType.DMA((NB,)),
        pltpu.SemaphoreType.DMA((NB,)),
        pltpu.SemaphoreType.DMA((2,)),
    ],
)


def _pool_body(h3tbl, batch2d, zg, out, bidx, rows, acc):
    cid = lax.axis_index("c")
    sid = lax.axis_index("s")
    gstripe = GP // 16
    pltpu.sync_copy(zg.at[pl.ds(sid * gstripe, gstripe)],
                    acc.at[pl.ds(sid * gstripe, gstripe)])
    plsc.subcore_barrier()

    def step(c, carry):
        ch = sid * 25 + c
        pltpu.sync_copy(batch2d.at[pl.ds(ch, 1)], bidx)
        pltpu.sync_copy(h3tbl.at[pl.ds(cid * NP + ch * 128, 128)], rows)
        pltpu.sync_copy(rows, acc.at[bidx.at[0]], add=True)
        return carry

    lax.fori_loop(0, 25, step, 0)
    plsc.subcore_barrier()
    pltpu.sync_copy(acc.at[pl.ds(sid * gstripe, gstripe)],
                    out.at[pl.ds(cid * GP + sid * gstripe, gstripe)])


_sck_pool = pl.kernel(
    _pool_body,
    out_type=jax.ShapeDtypeStruct((2 * GP, HC), jnp.float32),
    mesh=_mesh,
    compiler_params=_sc_params,
    scratch_types=[
        pltpu.VMEM((1, 128), jnp.int32),
        pltpu.VMEM((128, HC), jnp.float32),
        pltpu.VMEM_SHARED((GP, HC), jnp.float32),
    ],
)


R = 512


def _tck1_body(dp_ref, xp_ref, dinv_ref, g1_ref):
    deg = jnp.sum(dp_ref[...], axis=0) + 1.0
    dv = lax.rsqrt(deg)
    dinv_ref[...] = dv[:, None]
    g1_ref[...] = xp_ref[...] * dv[:, None]


def _tck1(deg_parts, xp):
    return pl.pallas_call(
        _tck1_body,
        grid=(NP // R,),
        in_specs=[
            pl.BlockSpec((32, R), lambda i: (0, i)),
            pl.BlockSpec((R, 16), lambda i: (i, 0)),
        ],
        out_specs=[
            pl.BlockSpec((R, 1), lambda i: (i, 0)),
            pl.BlockSpec((R, 16), lambda i: (i, 0)),
        ],
        out_shape=[
            jax.ShapeDtypeStruct((NP, 1), jnp.float32),
            jax.ShapeDtypeStruct((NP, 16), jnp.float32),
        ],
    )(deg_parts, xp)


def _tck2_body(ag_ref, dv_ref, w_ref, b_ref, o_ref):
    a = ag_ref[0] + ag_ref[1]
    dv = dv_ref[...]
    h = jnp.maximum(
        jnp.dot(a * dv, w_ref[...], preferred_element_type=jnp.float32)
        + b_ref[...], 0.0)
    g = h * dv
    o_ref[0] = g[:, :HC]
    o_ref[1] = g[:, HC:]


def _tck2(agg1, dinv, w1p, b1p):
    return pl.pallas_call(
        _tck2_body,
        grid=(NP // R,),
        in_specs=[
            pl.BlockSpec((2, R, 16), lambda i: (0, i, 0)),
            pl.BlockSpec((R, 1), lambda i: (i, 0)),
            pl.BlockSpec((16, DP), lambda i: (0, 0)),
            pl.BlockSpec((1, DP), lambda i: (0, 0)),
        ],
        out_specs=pl.BlockSpec((2, R, HC), lambda i: (0, i, 0)),
        out_shape=jax.ShapeDtypeStruct((2, NP, HC), jnp.float32),
    )(agg1, dinv, w1p, b1p)


def _tck3_body(final, ag_ref, dv_ref, w_ref, b_ref, o_ref):
    hin = jnp.concatenate([ag_ref[0], ag_ref[1]], axis=1)
    dv = dv_ref[...]
    h = jnp.maximum(
        jnp.dot(hin * dv, w_ref[...], preferred_element_type=jnp.float32)
        + b_ref[...], 0.0)
    g = h if final else h * dv
    o_ref[0] = g[:, :HC]
    o_ref[1] = g[:, HC:]


def _tck3(agg, dinv, wp, bp, final):
    return pl.pallas_call(
        functools.partial(_tck3_body, final),
        grid=(NP // R,),
        in_specs=[
            pl.BlockSpec((2, R, HC), lambda i: (0, i, 0)),
            pl.BlockSpec((R, 1), lambda i: (i, 0)),
            pl.BlockSpec((DP, DP), lambda i: (0, 0)),
            pl.BlockSpec((1, DP), lambda i: (0, 0)),
        ],
        out_specs=pl.BlockSpec((2, R, HC), lambda i: (0, i, 0)),
        out_shape=jax.ShapeDtypeStruct((2, NP, HC), jnp.float32),
    )(agg, dinv, wp, bp)


def _head_body(pool_ref, cp_ref, bn2_ref, linw_ref, linb_ref, bn3_ref,
               lin2w_ref, lin2b_ref, lin3w_ref, lin3b_ref, o_ref):
    counts = jnp.sum(cp_ref[...], axis=0)[:, None]
    invc = 1.0 / jnp.maximum(counts, 1.0)
    hg = jnp.concatenate([pool_ref[0], pool_ref[1]], axis=1) * invc
    bn2 = bn2_ref[...]
    s2 = bn2[0] * lax.rsqrt(bn2[3] + EPS)
    hg = (hg - bn2[2]) * s2 + bn2[1]
    h = jnp.maximum(
        jnp.dot(hg, linw_ref[...], preferred_element_type=jnp.float32)
        + linb_ref[...], 0.0)
    bn3 = bn3_ref[...]
    s3 = bn3[0] * lax.rsqrt(bn3[3] + EPS)
    h = (h - bn3[2]) * s3 + bn3[1]
    h = jnp.maximum(
        jnp.dot(h, lin2w_ref[...], preferred_element_type=jnp.float32)
        + lin2b_ref[...], 0.0)
    o = (jnp.dot(h, lin3w_ref[...], preferred_element_type=jnp.float32)
         + lin3b_ref[...])
    m = jnp.max(o, axis=1, keepdims=True)
    z = o - m
    lse = jnp.log(jnp.sum(jnp.exp(z), axis=1, keepdims=True))
    o_ref[...] = z - lse


def _tck_head(pool, cparts, bn2, linw, linb, bn3, lin2w, lin2b, lin3w, lin3b):
    full = lambda s: pl.BlockSpec(s, lambda: tuple(0 for _ in s))
    return pl.pallas_call(
        _head_body,
        in_specs=[
            full((2, G, HC)), full((16, G)), full((4, DP)), full((DP, DP)),
            full((1, DP)), full((4, DP)), full((DP, DP)), full((1, DP)),
            full((DP, DP)), full((1, DP)),
        ],
        out_specs=full((G, DP)),
        out_shape=jax.ShapeDtypeStruct((G, DP), jnp.float32),
    )(pool[:, :G, :], cparts[:, :G], bn2, linw, linb, bn3, lin2w, lin2b,
      lin3w, lin3b)


def _pad_vec(v, fill=0.0):
    return jnp.pad(v, (0, DP - v.shape[0]), constant_values=fill)[None, :]


def kernel(x, edge_index, batch, W1, b1, W2, b2, W3, b3, bn2_g, bn2_b, bn2_m,
           bn2_v, bn3_g, bn3_b, bn3_m, bn3_v, lin_W, lin_b, lin2_W, lin2_b,
           lin3_W, lin3_b):
    src = jnp.concatenate(
        [edge_index[0], jnp.full((EP - E,), N, jnp.int32)]).reshape(NCH, 128)
    dst = jnp.concatenate(
        [edge_index[1], jnp.full((EP - E,), N, jnp.int32)]).reshape(NCH, 128)
    srcp = jnp.stack([src, src + NP])
    batch2d = jnp.concatenate(
        [batch, jnp.full((NP - N,), G, jnp.int32)]).reshape(NP // 128, 128)
    xp = jnp.pad(x, ((0, NP - N), (0, 0)))
    znp = jnp.zeros((NP,), jnp.float32)
    zt16 = jnp.zeros((NP, 16), jnp.float32)
    zg = jnp.zeros((GP, HC), jnp.float32)
    w1p = jnp.pad(W1, ((0, 0), (0, DP - D)))
    w2p = jnp.pad(W2, ((0, DP - D), (0, DP - D)))
    w3p = jnp.pad(W3, ((0, DP - D), (0, DP - D)))
    linwp = jnp.pad(lin_W, ((0, DP - D), (0, DP - D)))
    lin2wp = jnp.pad(lin2_W, ((0, DP - D), (0, DP - D)))
    lin3wp = jnp.pad(lin3_W, ((0, DP - D), (0, DP - 4)))
    b1p, b2p, b3p = _pad_vec(b1), _pad_vec(b2), _pad_vec(b3)
    linbp, lin2bp = _pad_vec(lin_b), _pad_vec(lin2_b)
    lin3bp = _pad_vec(lin3_b, fill=-1e30)
    bn2 = jnp.stack([
        _pad_vec(bn2_g)[0], _pad_vec(bn2_b)[0], _pad_vec(bn2_m)[0],
        _pad_vec(bn2_v, fill=1.0)[0]])
    bn3 = jnp.stack([
        _pad_vec(bn3_g)[0], _pad_vec(bn3_b)[0], _pad_vec(bn3_m)[0],
        _pad_vec(bn3_v, fill=1.0)[0]])

    deg_parts, cnt_parts = _sck_deg(dst, batch2d, znp)
    dinv, g1 = _tck1(deg_parts, xp)

    agg1 = _sck_agg16(g1, zt16, src, dst).reshape(2, NP, 16)
    g2 = _tck2(agg1, dinv, w1p, b1p)
    agg2 = _sck_agg32(g2.reshape(2 * NP, HC), srcp, dst).reshape(2, NP, HC)
    g3 = _tck3(agg2, dinv, w2p, b2p, final=False)
    agg3 = _sck_agg32(g3.reshape(2 * NP, HC), srcp, dst).reshape(2, NP, HC)
    h3 = _tck3(agg3, dinv, w3p, b3p, final=True)

    pool = _sck_pool(h3.reshape(2 * NP, HC), batch2d, zg).reshape(2, GP, HC)
    out = _tck_head(pool, cnt_parts, bn2, linwp, linbp, bn3, lin2wp, lin2bp,
                    lin3wp, lin3bp)
    return out[:, :4]


# speedup vs baseline: 1.2671x; 1.0343x over previous
import functools

import jax
import jax.numpy as jnp
from jax import lax
from jax.experimental import pallas as pl
from jax.experimental.pallas import tpu as pltpu
from jax.experimental.pallas import tpu_sc as plsc

N = 50000
NP = 51200
E = 1600000
EP = 1605632
NCH = EP // 128
G = 512
GP = 1024
D = 60
DP = 64
HC = 32
SK = 8
NB = 5
LA = 2
EPS = 1e-5

_mesh = plsc.VectorSubcoreMesh(core_axis_name="c", subcore_axis_name="s")
_sc_params = pltpu.CompilerParams(needs_layout_passes=False,
                                  use_tc_tiling_on_sc=False)


def _deg_body(dst2d, batch2d, znp, deg_out, cnt_out, didx, bidx, deg_acc,
              cnt_acc, isem):
    cid = lax.axis_index("c")
    sid = lax.axis_index("s")
    wid = cid * 16 + sid
    ones16 = jnp.ones((16,), jnp.float32)
    pltpu.sync_copy(znp.at[pl.ds(0, NP)], deg_acc)

    cpt = NCH // 32
    nsteps = cpt // SK

    def fire_idx(s):
        q = s % 2
        pltpu.async_copy(dst2d.at[pl.ds(wid * cpt + s * SK, SK)], didx.at[q],
                         isem.at[q])

    def step(s, carry):
        q = s % 2

        @pl.when(s + 1 < nsteps)
        def _():
            fire_idx(s + 1)

        pltpu.make_async_copy(dst2d.at[pl.ds(0, SK)], didx.at[q],
                              isem.at[q]).wait()
        for j in range(SK):
            for k in range(8):
                vec = didx[q, j, pl.ds(k * 16, 16)]
                plsc.addupdate_scatter(deg_acc, [vec], ones16)
        return carry

    fire_idx(0)
    lax.fori_loop(0, nsteps, step, 0)
    pltpu.sync_copy(deg_acc, deg_out.at[wid])

    @pl.when(cid == 0)
    def _():
        pltpu.sync_copy(znp.at[pl.ds(0, GP)], cnt_acc)
        pltpu.sync_copy(batch2d.at[pl.ds(sid * 25, 25)], bidx)
        for c in range(25):
            for k in range(8):
                vec = bidx[c, pl.ds(k * 16, 16)]
                plsc.addupdate_scatter(cnt_acc, [vec], ones16)
        pltpu.sync_copy(cnt_acc, cnt_out.at[sid])


_sck_deg = pl.kernel(
    _deg_body,
    out_type=(jax.ShapeDtypeStruct((32, NP), jnp.float32),
              jax.ShapeDtypeStruct((16, GP), jnp.float32)),
    mesh=_mesh,
    compiler_params=_sc_params,
    scratch_types=[
        pltpu.VMEM((2, SK, 128), jnp.int32),
        pltpu.VMEM((25, 128), jnp.int32),
        pltpu.VMEM((NP,), jnp.float32),
        pltpu.VMEM((GP,), jnp.float32),
        pltpu.SemaphoreType.DMA((2,)),
    ],
)


def _agg_loop(table, sidx_src, dst2d, rows, sidx, didx, gsem, ssem, isem,
              acc, cpt, base_of):
    nsteps = cpt // SK

    def drain(s, b, guarded):
        w = lambda: pltpu.make_async_copy(
            rows.at[b], acc.at[didx.at[0, 0]], ssem.at[b]).wait()
        if guarded:
            pl.when(s > 0)(w)
        else:
            w()

    def fire_idx(s):
        q = s % 2
        pltpu.async_copy(sidx_src(base_of(s)), sidx.at[q], isem.at[q])
        pltpu.async_copy(dst2d.at[pl.ds(base_of(s), SK)], didx.at[s % 3],
                         isem.at[q])

    def fire_gather(p2, j, b):
        pltpu.async_copy(table.at[sidx.at[p2, j]], rows.at[b], gsem.at[b])

    def step(s, carry):
        p2 = s % 2
        p3 = s % 3

        @pl.when(s + 1 < nsteps)
        def _():
            fire_idx(s + 1)

        pltpu.make_async_copy(sidx_src(base_of(s)), sidx.at[p2],
                              isem.at[p2]).wait()
        pltpu.make_async_copy(dst2d.at[pl.ds(base_of(s), SK)], didx.at[p3],
                              isem.at[p2]).wait()
        for j in range(LA + 1):
            drain(s, j % NB, True)
            fire_gather(p2, j, j % NB)
        for j in range(SK):
            b = j % NB
            nj = j + LA + 1
            if nj < SK:
                b1 = nj % NB
                drain(s, b1, nj < NB)
                fire_gather(p2, nj, b1)
            pltpu.make_async_copy(
                table.at[sidx.at[p2, j]], rows.at[b], gsem.at[b]).wait()
            pltpu.async_copy(rows.at[b], acc.at[didx.at[p3, j]], ssem.at[b],
                             add=True)
        return carry

    fire_idx(0)
    lax.fori_loop(0, nsteps, step, 0)
    for b in range(NB):
        pltpu.make_async_copy(rows.at[b], acc.at[didx.at[0, 0]],
                              ssem.at[b]).wait()


def _agg16_body(table, zt, src2d, dst2d, out, sidx, didx, rows, acc, gsem,
                ssem, isem):
    cid = lax.axis_index("c")
    sid = lax.axis_index("s")
    wid = cid * 16 + sid
    stripe = NP // 16

    @pl.when(cid == 0)
    def _():
        pltpu.sync_copy(table.at[pl.ds(sid * stripe, stripe)],
                        acc.at[pl.ds(sid * stripe, stripe)])

    @pl.when(cid == 1)
    def _():
        pltpu.sync_copy(zt.at[pl.ds(sid * stripe, stripe)],
                        acc.at[pl.ds(sid * stripe, stripe)])

    plsc.subcore_barrier()
    cpt = NCH // 32
    _agg_loop(table, lambda base: src2d.at[pl.ds(base, SK)], dst2d, rows,
              sidx, didx, gsem, ssem, isem, acc, cpt,
              lambda s: wid * cpt + s * SK)
    plsc.subcore_barrier()
    pltpu.sync_copy(acc.at[pl.ds(sid * stripe, stripe)],
                    out.at[pl.ds(cid * NP + sid * stripe, stripe)])


_sck_agg16 = pl.kernel(
    _agg16_body,
    out_type=jax.ShapeDtypeStruct((2 * NP, 16), jnp.float32),
    mesh=_mesh,
    compiler_params=_sc_params,
    scratch_types=[
        pltpu.VMEM((2, SK, 128), jnp.int32),
        pltpu.VMEM((3, SK, 128), jnp.int32),
        pltpu.VMEM((NB, 128, 16), jnp.float32),
        pltpu.VMEM_SHARED((NP, 16), jnp.float32),
        pltpu.SemaphoreType.DMA((NB,)),
        pltpu.SemaphoreType.DMA((NB,)),
        pltpu.SemaphoreType.DMA((2,)),
    ],
)


def _agg32_body(table, srcp, dst2d, out, sidx, didx, rows, acc, gsem, ssem,
                isem):
    cid = lax.axis_index("c")
    sid = lax.axis_index("s")
    stripe = NP // 16
    pltpu.sync_copy(table.at[pl.ds(cid * NP + sid * stripe, stripe)],
                    acc.at[pl.ds(sid * stripe, stripe)])
    plsc.subcore_barrier()
    cpt = NCH // 16
    _agg_loop(table, lambda base: srcp.at[cid, pl.ds(base, SK)], dst2d, rows,
              sidx, didx, gsem, ssem, isem, acc, cpt,
              lambda s: sid * cpt + s * SK)
    plsc.subcore_barrier()
    pltpu.sync_copy(acc.at[pl.ds(sid * stripe, stripe)],
                    out.at[pl.ds(cid * NP + sid * stripe, stripe)])


_sck_agg32 = pl.kernel(
    _agg32_body,
    out_type=jax.ShapeDtypeStruct((2 * NP, HC), jnp.float32),
    mesh=_mesh,
    compiler_params=_sc_params,
    scratch_types=[
        pltpu.VMEM((2, SK, 128), jnp.int32),
        pltpu.VMEM((3, SK, 128), jnp.int32),
        pltpu.VMEM((NB, 128, HC), jnp.float32),
        pltpu.VMEM_SHARED((NP, HC), jnp.float32),
        pltpu.SemaphoreType.DMA((NB,)),
        pltpu.SemaphoreType.DMA((NB,)),
        pltpu.SemaphoreType.DMA((2,)),
    ],
)


def _pool_body(h3tbl, batch2d, zg, out, bidx, rows, acc, rsem):
    cid = lax.axis_index("c")
    sid = lax.axis_index("s")
    gstripe = GP // 16
    pltpu.sync_copy(zg.at[pl.ds(sid * gstripe, gstripe)],
                    acc.at[pl.ds(sid * gstripe, gstripe)])
    pltpu.sync_copy(batch2d.at[pl.ds(sid * 25, 25)], bidx)
    plsc.subcore_barrier()

    def fire_rows(c):
        q = c % 2
        pltpu.async_copy(h3tbl.at[pl.ds(cid * NP + (sid * 25 + c) * 128, 128)],
                         rows.at[q], rsem.at[q])

    def step(c, carry):
        q = c % 2

        @pl.when(c + 1 < 25)
        def _():
            fire_rows(c + 1)

        pltpu.make_async_copy(h3tbl.at[pl.ds(0, 128)], rows.at[q],
                              rsem.at[q]).wait()
        pltpu.sync_copy(rows.at[q], acc.at[bidx.at[c]], add=True)
        return carry

    fire_rows(0)
    lax.fori_loop(0, 25, step, 0)
    plsc.subcore_barrier()
    pltpu.sync_copy(acc.at[pl.ds(sid * gstripe, gstripe)],
                    out.at[pl.ds(cid * GP + sid * gstripe, gstripe)])


_sck_pool = pl.kernel(
    _pool_body,
    out_type=jax.ShapeDtypeStruct((2 * GP, HC), jnp.float32),
    mesh=_mesh,
    compiler_params=_sc_params,
    scratch_types=[
        pltpu.VMEM((25, 128), jnp.int32),
        pltpu.VMEM((2, 128, HC), jnp.float32),
        pltpu.VMEM_SHARED((GP, HC), jnp.float32),
        pltpu.SemaphoreType.DMA((2,)),
    ],
)


R = 512


def _tck1_body(dp_ref, xp_ref, dinv_ref, g1_ref):
    deg = jnp.sum(dp_ref[...], axis=0) + 1.0
    dv = lax.rsqrt(deg)
    dinv_ref[...] = dv[:, None]
    g1_ref[...] = xp_ref[...] * dv[:, None]


def _tck1(deg_parts, xp):
    return pl.pallas_call(
        _tck1_body,
        grid=(NP // R,),
        in_specs=[
            pl.BlockSpec((32, R), lambda i: (0, i)),
            pl.BlockSpec((R, 16), lambda i: (i, 0)),
        ],
        out_specs=[
            pl.BlockSpec((R, 1), lambda i: (i, 0)),
            pl.BlockSpec((R, 16), lambda i: (i, 0)),
        ],
        out_shape=[
            jax.ShapeDtypeStruct((NP, 1), jnp.float32),
            jax.ShapeDtypeStruct((NP, 16), jnp.float32),
        ],
    )(deg_parts, xp)


def _tck2_body(ag_ref, dv_ref, w_ref, b_ref, o_ref):
    a = ag_ref[0] + ag_ref[1]
    dv = dv_ref[...]
    h = jnp.maximum(
        jnp.dot(a * dv, w_ref[...], preferred_element_type=jnp.float32)
        + b_ref[...], 0.0)
    g = h * dv
    o_ref[0] = g[:, :HC]
    o_ref[1] = g[:, HC:]


def _tck2(agg1, dinv, w1p, b1p):
    return pl.pallas_call(
        _tck2_body,
        grid=(NP // R,),
        in_specs=[
            pl.BlockSpec((2, R, 16), lambda i: (0, i, 0)),
            pl.BlockSpec((R, 1), lambda i: (i, 0)),
            pl.BlockSpec((16, DP), lambda i: (0, 0)),
            pl.BlockSpec((1, DP), lambda i: (0, 0)),
        ],
        out_specs=pl.BlockSpec((2, R, HC), lambda i: (0, i, 0)),
        out_shape=jax.ShapeDtypeStruct((2, NP, HC), jnp.float32),
    )(agg1, dinv, w1p, b1p)


def _tck3_body(final, ag_ref, dv_ref, w_ref, b_ref, o_ref):
    hin = jnp.concatenate([ag_ref[0], ag_ref[1]], axis=1)
    dv = dv_ref[...]
    h = jnp.maximum(
        jnp.dot(hin * dv, w_ref[...], preferred_element_type=jnp.float32)
        + b_ref[...], 0.0)
    g = h if final else h * dv
    o_ref[0] = g[:, :HC]
    o_ref[1] = g[:, HC:]


def _tck3(agg, dinv, wp, bp, final):
    return pl.pallas_call(
        functools.partial(_tck3_body, final),
        grid=(NP // R,),
        in_specs=[
            pl.BlockSpec((2, R, HC), lambda i: (0, i, 0)),
            pl.BlockSpec((R, 1), lambda i: (i, 0)),
            pl.BlockSpec((DP, DP), lambda i: (0, 0)),
            pl.BlockSpec((1, DP), lambda i: (0, 0)),
        ],
        out_specs=pl.BlockSpec((2, R, HC), lambda i: (0, i, 0)),
        out_shape=jax.ShapeDtypeStruct((2, NP, HC), jnp.float32),
    )(agg, dinv, wp, bp)


def _head_body(pool_ref, cp_ref, bn2_ref, linw_ref, linb_ref, bn3_ref,
               lin2w_ref, lin2b_ref, lin3w_ref, lin3b_ref, o_ref):
    counts = jnp.sum(cp_ref[...], axis=0)[:, None]
    invc = 1.0 / jnp.maximum(counts, 1.0)
    hg = jnp.concatenate([pool_ref[0], pool_ref[1]], axis=1) * invc
    bn2 = bn2_ref[...]
    s2 = bn2[0] * lax.rsqrt(bn2[3] + EPS)
    hg = (hg - bn2[2]) * s2 + bn2[1]
    h = jnp.maximum(
        jnp.dot(hg, linw_ref[...], preferred_element_type=jnp.float32)
        + linb_ref[...], 0.0)
    bn3 = bn3_ref[...]
    s3 = bn3[0] * lax.rsqrt(bn3[3] + EPS)
    h = (h - bn3[2]) * s3 + bn3[1]
    h = jnp.maximum(
        jnp.dot(h, lin2w_ref[...], preferred_element_type=jnp.float32)
        + lin2b_ref[...], 0.0)
    o = (jnp.dot(h, lin3w_ref[...], preferred_element_type=jnp.float32)
         + lin3b_ref[...])
    m = jnp.max(o, axis=1, keepdims=True)
    z = o - m
    lse = jnp.log(jnp.sum(jnp.exp(z), axis=1, keepdims=True))
    o_ref[...] = z - lse


def _tck_head(pool, cparts, bn2, linw, linb, bn3, lin2w, lin2b, lin3w, lin3b):
    full = lambda s: pl.BlockSpec(s, lambda: tuple(0 for _ in s))
    return pl.pallas_call(
        _head_body,
        in_specs=[
            full((2, G, HC)), full((16, G)), full((4, DP)), full((DP, DP)),
            full((1, DP)), full((4, DP)), full((DP, DP)), full((1, DP)),
            full((DP, DP)), full((1, DP)),
        ],
        out_specs=full((G, DP)),
        out_shape=jax.ShapeDtypeStruct((G, DP), jnp.float32),
    )(pool[:, :G, :], cparts[:, :G], bn2, linw, linb, bn3, lin2w, lin2b,
      lin3w, lin3b)


def _pad_vec(v, fill=0.0):
    return jnp.pad(v, (0, DP - v.shape[0]), constant_values=fill)[None, :]


def kernel(x, edge_index, batch, W1, b1, W2, b2, W3, b3, bn2_g, bn2_b, bn2_m,
           bn2_v, bn3_g, bn3_b, bn3_m, bn3_v, lin_W, lin_b, lin2_W, lin2_b,
           lin3_W, lin3_b):
    src = jnp.concatenate(
        [edge_index[0], jnp.full((EP - E,), N, jnp.int32)]).reshape(NCH, 128)
    dst = jnp.concatenate(
        [edge_index[1], jnp.full((EP - E,), N, jnp.int32)]).reshape(NCH, 128)
    srcp = jnp.stack([src, src + NP])
    batch2d = jnp.concatenate(
        [batch, jnp.full((NP - N,), G, jnp.int32)]).reshape(NP // 128, 128)
    xp = jnp.pad(x, ((0, NP - N), (0, 0)))
    znp = jnp.zeros((NP,), jnp.float32)
    zt16 = jnp.zeros((NP, 16), jnp.float32)
    zg = jnp.zeros((GP, HC), jnp.float32)
    w1p = jnp.pad(W1, ((0, 0), (0, DP - D)))
    w2p = jnp.pad(W2, ((0, DP - D), (0, DP - D)))
    w3p = jnp.pad(W3, ((0, DP - D), (0, DP - D)))
    linwp = jnp.pad(lin_W, ((0, DP - D), (0, DP - D)))
    lin2wp = jnp.pad(lin2_W, ((0, DP - D), (0, DP - D)))
    lin3wp = jnp.pad(lin3_W, ((0, DP - D), (0, DP - 4)))
    b1p, b2p, b3p = _pad_vec(b1), _pad_vec(b2), _pad_vec(b3)
    linbp, lin2bp = _pad_vec(lin_b), _pad_vec(lin2_b)
    lin3bp = _pad_vec(lin3_b, fill=-1e30)
    bn2 = jnp.stack([
        _pad_vec(bn2_g)[0], _pad_vec(bn2_b)[0], _pad_vec(bn2_m)[0],
        _pad_vec(bn2_v, fill=1.0)[0]])
    bn3 = jnp.stack([
        _pad_vec(bn3_g)[0], _pad_vec(bn3_b)[0], _pad_vec(bn3_m)[0],
        _pad_vec(bn3_v, fill=1.0)[0]])

    deg_parts, cnt_parts = _sck_deg(dst, batch2d, znp)
    dinv, g1 = _tck1(deg_parts, xp)

    agg1 = _sck_agg16(g1, zt16, src, dst).reshape(2, NP, 16)
    g2 = _tck2(agg1, dinv, w1p, b1p)
    agg2 = _sck_agg32(g2.reshape(2 * NP, HC), srcp, dst).reshape(2, NP, HC)
    g3 = _tck3(agg2, dinv, w2p, b2p, final=False)
    agg3 = _sck_agg32(g3.reshape(2 * NP, HC), srcp, dst).reshape(2, NP, HC)
    h3 = _tck3(agg3, dinv, w3p, b3p, final=True)

    pool = _sck_pool(h3.reshape(2 * NP, HC), batch2d, zg).reshape(2, GP, HC)
    out = _tck_head(pool, cnt_parts, bn2, linwp, linbp, bn3, lin2wp, lin2bp,
                    lin3wp, lin3bp)
    return out[:, :4]
